# baseline (speedup 1.0000x reference)
import jax
import jax.numpy as jnp
from jax import lax
from jax.experimental import pallas as pl
from jax.experimental.pallas import tpu as pltpu
from jax.experimental.pallas import tpu_sc as plsc

NUM_PROMPTS = 64
PROMPT_DIM = 1024
PROMPT_LENGTH = 4
KEY_DIM = 1024
TOP_K = 8
INV_TEMP = 1.0 / (1.0 + 1e-8)

B_TOTAL = 8192
N_CHUNKS = 1
B_CHUNK = B_TOTAL // N_CHUNKS
SIM_BLK = 1024
OUT_BLK = 256

SC_CORES = 2
SC_SUBCORES = 16
SC_WORKERS = SC_CORES * SC_SUBCORES
COLS_PER_W = B_CHUNK // SC_WORKERS
LANES = 16
GROUPS = COLS_PER_W // LANES


def _sim_body(q_ref, k_ref, s_ref):
    k = k_ref[...]
    kn = k / jnp.maximum(
        jnp.sqrt(jnp.sum(k * k, axis=-1, keepdims=True)), 1e-12)
    q = q_ref[...]
    qn = q / jnp.maximum(
        jnp.sqrt(jnp.sum(q * q, axis=-1, keepdims=True)), 1e-12)
    s_ref[...] = lax.dot_general(kn, qn, (((1,), (1,)), ((), ())),
                                 preferred_element_type=jnp.float32)


def _out_body(wt_ref, p_ref, o_ref):
    o_ref[...] = lax.dot_general(wt_ref[...], p_ref[...],
                                 (((0,), (0,)), ((), ())),
                                 preferred_element_type=jnp.float32)


def _route_body(simt_hbm, wt_hbm, sim_v, w_v):
    wid = lax.axis_index("s") * SC_CORES + lax.axis_index("c")
    base = wid * COLS_PER_W
    pltpu.sync_copy(simt_hbm.at[:, pl.ds(base, COLS_PER_W)], sim_v)

    def one_group(off):
        r = [jnp.full((LANES,), -jnp.inf, jnp.float32) for _ in range(TOP_K)]
        for n in range(NUM_PROMPTS):
            x = sim_v[n, pl.ds(off, LANES)]
            for i in range(TOP_K):
                hi = jnp.maximum(r[i], x)
                x = jnp.minimum(r[i], x)
                r[i] = hi
        t = r[TOP_K - 1]
        mx = r[0]
        denom = jnp.full((LANES,), 0.0, jnp.float32)
        cgt = jnp.full((LANES,), 0, jnp.int32)
        one = jnp.full((LANES,), 1, jnp.int32)
        zero = jnp.full((LANES,), 0, jnp.int32)
        for i in range(TOP_K):
            denom = denom + jnp.exp((r[i] - mx) * INV_TEMP)
            cgt = cgt + jnp.where(r[i] > t, one, zero)
        recip = 1.0 / denom
        budget = TOP_K - cgt
        for n in range(NUM_PROMPTS):
            x = sim_v[n, pl.ds(off, LANES)]
            gt = x > t
            take_eq = jnp.logical_and(x == t, budget > 0)
            budget = budget - jnp.where(take_eq, one, zero)
            take = jnp.logical_or(gt, take_eq)
            w_v[n, pl.ds(off, LANES)] = jnp.where(
                take, jnp.exp((x - mx) * INV_TEMP) * recip, 0.0)

    def group(g, carry):
        one_group(g * LANES)
        return carry

    lax.fori_loop(0, GROUPS, group, 0)
    pltpu.sync_copy(w_v, wt_hbm.at[:, pl.ds(base, COLS_PER_W)])


_route = pl.kernel(
    _route_body,
    out_type=jax.ShapeDtypeStruct((NUM_PROMPTS, B_CHUNK), jnp.float32),
    mesh=plsc.VectorSubcoreMesh(core_axis_name="c", subcore_axis_name="s",
                                num_cores=SC_CORES,
                                num_subcores=SC_SUBCORES),
    scratch_types=[
        pltpu.VMEM((NUM_PROMPTS, COLS_PER_W), jnp.float32),
        pltpu.VMEM((NUM_PROMPTS, COLS_PER_W), jnp.float32),
    ],
)


@jax.jit
def kernel(query, prompts, keys):
    B = query.shape[0]
    p_flat = prompts.reshape(NUM_PROMPTS, PROMPT_LENGTH * PROMPT_DIM)

    def sim_chunk(q_chunk):
        return pl.pallas_call(
            _sim_body,
            grid=(B_CHUNK // SIM_BLK,),
            in_specs=[
                pl.BlockSpec((SIM_BLK, KEY_DIM), lambda i: (i, 0)),
                pl.BlockSpec((NUM_PROMPTS, KEY_DIM), lambda i: (0, 0)),
            ],
            out_specs=pl.BlockSpec((NUM_PROMPTS, SIM_BLK), lambda i: (0, i)),
            out_shape=jax.ShapeDtypeStruct((NUM_PROMPTS, B_CHUNK),
                                           jnp.float32),
        )(q_chunk, keys)

    wts = [_route(sim_chunk(query[c * B_CHUNK:(c + 1) * B_CHUNK]))
           for c in range(N_CHUNKS)]
    wt = jnp.concatenate(wts, axis=1)

    out = pl.pallas_call(
        _out_body,
        grid=(B // OUT_BLK,),
        in_specs=[
            pl.BlockSpec((NUM_PROMPTS, OUT_BLK), lambda i: (0, i)),
            pl.BlockSpec((NUM_PROMPTS, PROMPT_LENGTH * PROMPT_DIM),
                         lambda i: (0, 0)),
        ],
        out_specs=pl.BlockSpec((OUT_BLK, PROMPT_LENGTH * PROMPT_DIM),
                               lambda i: (i, 0)),
        out_shape=jax.ShapeDtypeStruct((B, PROMPT_LENGTH * PROMPT_DIM),
                                       jnp.float32),
    )(wt, p_flat)
    return out.reshape(B, PROMPT_LENGTH, PROMPT_DIM)


# speedup vs baseline: 1.0792x; 1.0792x over previous
import jax
import jax.numpy as jnp
from jax import lax
from jax.experimental import pallas as pl
from jax.experimental.pallas import tpu as pltpu
from jax.experimental.pallas import tpu_sc as plsc

NUM_PROMPTS = 64
PROMPT_DIM = 1024
PROMPT_LENGTH = 4
KEY_DIM = 1024
TOP_K = 8
INV_TEMP = 1.0 / (1.0 + 1e-8)

B_TOTAL = 8192
N_CHUNKS = 1
B_CHUNK = B_TOTAL // N_CHUNKS
SIM_BLK = 1024
OUT_BLK = 512

SC_CORES = 2
SC_SUBCORES = 16
SC_WORKERS = SC_CORES * SC_SUBCORES
COLS_PER_W = B_CHUNK // SC_WORKERS
LANES = 16
GROUPS = COLS_PER_W // LANES


def _sim_body(q_ref, k_ref, s_ref):
    k = k_ref[...]
    kn = k / jnp.maximum(
        jnp.sqrt(jnp.sum(k * k, axis=-1, keepdims=True)), 1e-12)
    q = q_ref[...]
    qn = q / jnp.maximum(
        jnp.sqrt(jnp.sum(q * q, axis=-1, keepdims=True)), 1e-12)
    s_ref[...] = lax.dot_general(kn, qn, (((1,), (1,)), ((), ())),
                                 preferred_element_type=jnp.float32)


def _out_body(st_ref, simt_ref, p_ref, o_ref):
    st = st_ref[...]
    t = st[0:1, :]
    mx = st[1:2, :]
    recip = st[2:3, :]
    budget = st[3:4, :]
    s = simt_ref[...]
    gt = s > t
    eq = s == t
    eqf = jnp.where(eq, 1.0, 0.0)
    pref = eqf
    k = 1
    while k < NUM_PROMPTS:
        shifted = jnp.concatenate(
            [jnp.zeros((k, s.shape[1]), jnp.float32), pref[:-k, :]], axis=0)
        pref = pref + shifted
        k *= 2
    excl = pref - eqf
    take = jnp.logical_or(gt, jnp.logical_and(eq, excl < budget))
    w = jnp.where(take, jnp.exp((s - mx) * INV_TEMP) * recip, 0.0)
    o_ref[...] = lax.dot_general(w, p_ref[...],
                                 (((0,), (0,)), ((), ())),
                                 preferred_element_type=jnp.float32)


def _route_body(simt_hbm, st_hbm, sim_v, st_v):
    wid = lax.axis_index("s") * SC_CORES + lax.axis_index("c")
    base = wid * COLS_PER_W
    pltpu.sync_copy(simt_hbm.at[:, pl.ds(base, COLS_PER_W)], sim_v)

    def one_group(off):
        r = [jnp.full((LANES,), -jnp.inf, jnp.float32) for _ in range(TOP_K)]
        for n in range(NUM_PROMPTS):
            x = sim_v[n, pl.ds(off, LANES)]
            for i in range(TOP_K):
                hi = jnp.maximum(r[i], x)
                x = jnp.minimum(r[i], x)
                r[i] = hi
        t = r[TOP_K - 1]
        mx = r[0]
        denom = jnp.full((LANES,), 0.0, jnp.float32)
        cgt = jnp.full((LANES,), 0.0, jnp.float32)
        one = jnp.full((LANES,), 1.0, jnp.float32)
        zero = jnp.full((LANES,), 0.0, jnp.float32)
        for i in range(TOP_K):
            denom = denom + jnp.exp((r[i] - mx) * INV_TEMP)
            cgt = cgt + jnp.where(r[i] > t, one, zero)
        st_v[0, pl.ds(off, LANES)] = t
        st_v[1, pl.ds(off, LANES)] = mx
        st_v[2, pl.ds(off, LANES)] = 1.0 / denom
        st_v[3, pl.ds(off, LANES)] = TOP_K - cgt

    def group(g, carry):
        one_group(g * LANES)
        return carry

    lax.fori_loop(0, GROUPS, group, 0)
    pltpu.sync_copy(st_v, st_hbm.at[:, pl.ds(base, COLS_PER_W)])


_route = pl.kernel(
    _route_body,
    out_type=jax.ShapeDtypeStruct((8, B_CHUNK), jnp.float32),
    mesh=plsc.VectorSubcoreMesh(core_axis_name="c", subcore_axis_name="s",
                                num_cores=SC_CORES,
                                num_subcores=SC_SUBCORES),
    scratch_types=[
        pltpu.VMEM((NUM_PROMPTS, COLS_PER_W), jnp.float32),
        pltpu.VMEM((8, COLS_PER_W), jnp.float32),
    ],
)


@jax.jit
def kernel(query, prompts, keys):
    B = query.shape[0]
    p_flat = prompts.reshape(NUM_PROMPTS, PROMPT_LENGTH * PROMPT_DIM)

    def sim_chunk(q_chunk):
        return pl.pallas_call(
            _sim_body,
            grid=(B_CHUNK // SIM_BLK,),
            in_specs=[
                pl.BlockSpec((SIM_BLK, KEY_DIM), lambda i: (i, 0)),
                pl.BlockSpec((NUM_PROMPTS, KEY_DIM), lambda i: (0, 0)),
            ],
            out_specs=pl.BlockSpec((NUM_PROMPTS, SIM_BLK), lambda i: (0, i)),
            out_shape=jax.ShapeDtypeStruct((NUM_PROMPTS, B_CHUNK),
                                           jnp.float32),
        )(q_chunk, keys)

    simt = sim_chunk(query)
    stats = _route(simt)

    out = pl.pallas_call(
        _out_body,
        grid=(B // OUT_BLK,),
        in_specs=[
            pl.BlockSpec((8, OUT_BLK), lambda i: (0, i)),
            pl.BlockSpec((NUM_PROMPTS, OUT_BLK), lambda i: (0, i)),
            pl.BlockSpec((NUM_PROMPTS, PROMPT_LENGTH * PROMPT_DIM),
                         lambda i: (0, 0)),
        ],
        out_specs=pl.BlockSpec((OUT_BLK, PROMPT_LENGTH * PROMPT_DIM),
                               lambda i: (i, 0)),
        out_shape=jax.ShapeDtypeStruct((B, PROMPT_LENGTH * PROMPT_DIM),
                                       jnp.float32),
    )(stats, simt, p_flat)
    return out.reshape(B, PROMPT_LENGTH, PROMPT_DIM)


# speedup vs baseline: 1.0891x; 1.0092x over previous
import jax
import jax.numpy as jnp
from jax import lax
from jax.experimental import pallas as pl
from jax.experimental.pallas import tpu as pltpu
from jax.experimental.pallas import tpu_sc as plsc

NUM_PROMPTS = 64
PROMPT_DIM = 1024
PROMPT_LENGTH = 4
KEY_DIM = 1024
TOP_K = 8
INV_TEMP = 1.0 / (1.0 + 1e-8)

B_TOTAL = 8192
N_CHUNKS = 1
B_CHUNK = B_TOTAL // N_CHUNKS
SIM_BLK = 2048
OUT_BLK = 512

SC_CORES = 2
SC_SUBCORES = 16
SC_WORKERS = SC_CORES * SC_SUBCORES
COLS_PER_W = B_CHUNK // SC_WORKERS
LANES = 16
GROUPS = COLS_PER_W // LANES


def _sim_body(q_ref, k_ref, s_ref):
    k = k_ref[...]
    kn = k / jnp.maximum(
        jnp.sqrt(jnp.sum(k * k, axis=-1, keepdims=True)), 1e-12)
    q = q_ref[...]
    qn = q / jnp.maximum(
        jnp.sqrt(jnp.sum(q * q, axis=-1, keepdims=True)), 1e-12)
    s_ref[...] = lax.dot_general(kn, qn, (((1,), (1,)), ((), ())),
                                 preferred_element_type=jnp.float32)


def _out_body(st_ref, simt_ref, p_ref, o_ref):
    st = st_ref[...]
    t = st[0:1, :]
    mx = st[1:2, :]
    recip = st[2:3, :]
    budget = st[3:4, :]
    s = simt_ref[...]
    gt = s > t
    eq = s == t
    eqf = jnp.where(eq, 1.0, 0.0)
    pref = eqf
    k = 1
    while k < NUM_PROMPTS:
        shifted = jnp.concatenate(
            [jnp.zeros((k, s.shape[1]), jnp.float32), pref[:-k, :]], axis=0)
        pref = pref + shifted
        k *= 2
    excl = pref - eqf
    take = jnp.logical_or(gt, jnp.logical_and(eq, excl < budget))
    w = jnp.where(take, jnp.exp((s - mx) * INV_TEMP) * recip, 0.0)
    o_ref[...] = lax.dot_general(w, p_ref[...],
                                 (((0,), (0,)), ((), ())),
                                 preferred_element_type=jnp.float32)


def _route_body(simt_hbm, st_hbm, sim_v, st_v):
    wid = lax.axis_index("s") * SC_CORES + lax.axis_index("c")
    base = wid * COLS_PER_W
    pltpu.sync_copy(simt_hbm.at[:, pl.ds(base, COLS_PER_W)], sim_v)

    def one_group(off):
        r = [jnp.full((LANES,), -jnp.inf, jnp.float32) for _ in range(TOP_K)]
        for n in range(NUM_PROMPTS):
            x = sim_v[n, pl.ds(off, LANES)]
            for i in range(TOP_K):
                hi = jnp.maximum(r[i], x)
                x = jnp.minimum(r[i], x)
                r[i] = hi
        t = r[TOP_K - 1]
        mx = r[0]
        denom = jnp.full((LANES,), 0.0, jnp.float32)
        cgt = jnp.full((LANES,), 0.0, jnp.float32)
        one = jnp.full((LANES,), 1.0, jnp.float32)
        zero = jnp.full((LANES,), 0.0, jnp.float32)
        for i in range(TOP_K):
            denom = denom + jnp.exp((r[i] - mx) * INV_TEMP)
            cgt = cgt + jnp.where(r[i] > t, one, zero)
        st_v[0, pl.ds(off, LANES)] = t
        st_v[1, pl.ds(off, LANES)] = mx
        st_v[2, pl.ds(off, LANES)] = 1.0 / denom
        st_v[3, pl.ds(off, LANES)] = TOP_K - cgt

    def group(g, carry):
        one_group(g * LANES)
        return carry

    lax.fori_loop(0, GROUPS, group, 0)
    pltpu.sync_copy(st_v, st_hbm.at[:, pl.ds(base, COLS_PER_W)])


_route = pl.kernel(
    _route_body,
    out_type=jax.ShapeDtypeStruct((8, B_CHUNK), jnp.float32),
    mesh=plsc.VectorSubcoreMesh(core_axis_name="c", subcore_axis_name="s",
                                num_cores=SC_CORES,
                                num_subcores=SC_SUBCORES),
    scratch_types=[
        pltpu.VMEM((NUM_PROMPTS, COLS_PER_W), jnp.float32),
        pltpu.VMEM((8, COLS_PER_W), jnp.float32),
    ],
)


@jax.jit
def kernel(query, prompts, keys):
    B = query.shape[0]
    p_flat = prompts.reshape(NUM_PROMPTS, PROMPT_LENGTH * PROMPT_DIM)

    def sim_chunk(q_chunk):
        return pl.pallas_call(
            _sim_body,
            grid=(B_CHUNK // SIM_BLK,),
            in_specs=[
                pl.BlockSpec((SIM_BLK, KEY_DIM), lambda i: (i, 0)),
                pl.BlockSpec((NUM_PROMPTS, KEY_DIM), lambda i: (0, 0)),
            ],
            out_specs=pl.BlockSpec((NUM_PROMPTS, SIM_BLK), lambda i: (0, i)),
            out_shape=jax.ShapeDtypeStruct((NUM_PROMPTS, B_CHUNK),
                                           jnp.float32),
        )(q_chunk, keys)

    simt = sim_chunk(query)
    stats = _route(simt)

    out = pl.pallas_call(
        _out_body,
        grid=(B // OUT_BLK,),
        in_specs=[
            pl.BlockSpec((8, OUT_BLK), lambda i: (0, i)),
            pl.BlockSpec((NUM_PROMPTS, OUT_BLK), lambda i: (0, i)),
            pl.BlockSpec((NUM_PROMPTS, PROMPT_LENGTH * PROMPT_DIM),
                         lambda i: (0, 0)),
        ],
        out_specs=pl.BlockSpec((OUT_BLK, PROMPT_LENGTH * PROMPT_DIM),
                               lambda i: (i, 0)),
        out_shape=jax.ShapeDtypeStruct((B, PROMPT_LENGTH * PROMPT_DIM),
                                       jnp.float32),
    )(stats, simt, p_flat)
    return out.reshape(B, PROMPT_LENGTH, PROMPT_DIM)
